# Initial kernel scaffold; baseline (speedup 1.0000x reference)
import jax
import jax.numpy as jnp
from jax.experimental import pallas as pl


def kernel(node_embeddings, source_indices, dest_indices, edge_types, num_edges, W_e, b_e, W_ir, b_ir, W_hr, W_iz, b_iz, W_hz, W_in, b_in, W_hn, b_hn):
    raise NotImplementedError("write your pallas kernel here")


# speedup vs baseline: 5.3207x; 5.3207x over previous
import functools

import jax
import jax.numpy as jnp
from jax import lax
from jax.experimental import pallas as pl
from jax.experimental.pallas import tpu as pltpu
from jax.experimental.pallas import tpu_sc as plsc

_H = 128
_NC = 2
_NS = 16
_CH = 128


def _dense_body(x_ref, w_ref, b_ref, o_ref):
    o_ref[0] = (
        jnp.dot(x_ref[...], w_ref[...], preferred_element_type=jnp.float32)
        + b_ref[0]
    )


def _edge_transform(x, w_cat, b_cat, nblk):
    n, h = x.shape
    g = w_cat.shape[1] // h
    ni = n // nblk
    return pl.pallas_call(
        _dense_body,
        grid=(ni, g),
        in_specs=[
            pl.BlockSpec((nblk, h), lambda i, t: (i, 0)),
            pl.BlockSpec((h, h), lambda i, t: (0, t)),
            pl.BlockSpec((1, 1, h), lambda i, t: (t, 0, 0)),
        ],
        out_specs=pl.BlockSpec((1, nblk, h), lambda i, t: (t, i, 0)),
        out_shape=jax.ShapeDtypeStruct((g, n, h), jnp.float32),
    )(x, w_cat, b_cat)


def _gru_body(part_ref, xr_ref, xz_ref, xn_ref, whr_ref, whz_ref, whn_ref,
              bhn_ref, o_ref):
    p = part_ref[0] + part_ref[1]
    r = jax.nn.sigmoid(
        xr_ref[0] + jnp.dot(p, whr_ref[...], preferred_element_type=jnp.float32))
    z = jax.nn.sigmoid(
        xz_ref[0] + jnp.dot(p, whz_ref[...], preferred_element_type=jnp.float32))
    nn = jnp.tanh(
        xn_ref[0]
        + r * (jnp.dot(p, whn_ref[...], preferred_element_type=jnp.float32)
               + bhn_ref[...]))
    o_ref[...] = (1.0 - z) * nn + z * p


def _gru(part, y, whr, whz, whn, bhn, nblk):
    npad = part.shape[1]
    n = y.shape[1]
    h = y.shape[2]
    ni = n // nblk
    return pl.pallas_call(
        _gru_body,
        grid=(ni,),
        in_specs=[
            pl.BlockSpec((2, nblk, h), lambda i: (0, i, 0)),
            pl.BlockSpec((1, nblk, h), lambda i: (6, i, 0)),
            pl.BlockSpec((1, nblk, h), lambda i: (7, i, 0)),
            pl.BlockSpec((1, nblk, h), lambda i: (8, i, 0)),
            pl.BlockSpec((h, h), lambda i: (0, 0)),
            pl.BlockSpec((h, h), lambda i: (0, 0)),
            pl.BlockSpec((h, h), lambda i: (0, 0)),
            pl.BlockSpec((1, h), lambda i: (0, 0)),
        ],
        out_specs=pl.BlockSpec((nblk, h), lambda i: (i, 0)),
        out_shape=jax.ShapeDtypeStruct((n, h), jnp.float32),
    )(part, y, y, y, whr, whz, whn, bhn)


def _sc_segment_sum(ytab, src, dst, typ, zrows, n_nodes, npad, rpt):
    e = src.shape[0]
    h = ytab.shape[1]
    nw = _NC * _NS
    epw = e // nw
    nsc = 20
    se = nsc * _CH
    nsup = -(-epw // se)

    mesh = plsc.VectorSubcoreMesh(core_axis_name="c", subcore_axis_name="s")

    @functools.partial(
        pl.kernel,
        mesh=mesh,
        out_type=jax.ShapeDtypeStruct((_NC, npad, h), jnp.float32),
        scratch_types=[
            pltpu.VMEM((se,), jnp.int32),
            pltpu.VMEM((se,), jnp.int32),
            pltpu.VMEM((se,), jnp.int32),
            pltpu.VMEM((nsc, _CH), jnp.int32),
            pltpu.VMEM((nsc, _CH), jnp.int32),
            pltpu.VMEM((_CH, h), jnp.float32),
            pltpu.VMEM((_CH, h), jnp.float32),
            pltpu.VMEM_SHARED((npad, h), jnp.float32),
            pltpu.SemaphoreType.DMA,
            pltpu.SemaphoreType.DMA,
        ],
    )
    def sck(ytab_h, src_h, dst_h, typ_h, z_h, out_h,
            sflat, tflat, dflat, gi2d, dj2d, rows0, rows1, acc, sem0, sem1):
        cid = lax.axis_index("c")
        sid = lax.axis_index("s")
        wid = cid * _NS + sid
        base = wid * epw

        pltpu.sync_copy(z_h, acc.at[pl.ds(sid * rpt, rpt)])
        plsc.subcore_barrier()

        zero16 = jnp.zeros((16,), jnp.int32)
        junk16 = jnp.full((16,), n_nodes, jnp.int32)

        for u in range(nsup):
            valid = min(se, epw - u * se)
            pltpu.sync_copy(src_h.at[pl.ds(base + u * se, valid)],
                            sflat.at[pl.ds(0, valid)])
            pltpu.sync_copy(typ_h.at[pl.ds(base + u * se, valid)],
                            tflat.at[pl.ds(0, valid)])
            pltpu.sync_copy(dst_h.at[pl.ds(base + u * se, valid)],
                            dflat.at[pl.ds(0, valid)])

            def cbody(j, carry):
                for k in range(_CH // 16):
                    off = j * _CH + k * 16
                    s = sflat[pl.ds(off, 16)]
                    t = tflat[pl.ds(off, 16)]
                    gi2d[j, pl.ds(k * 16, 16)] = t * n_nodes + s
                    dj2d[j, pl.ds(k * 16, 16)] = dflat[pl.ds(off, 16)]
                return carry
            lax.fori_loop(0, nsc, cbody, 0)

            for m in range(valid // 16, se // 16):
                j, k = m // (_CH // 16), m % (_CH // 16)
                gi2d[j, pl.ds(k * 16, 16)] = zero16
                dj2d[j, pl.ds(k * 16, 16)] = junk16

            def pbody(i, carry):
                a = i * 2
                b = a + 1
                ha = pltpu.async_copy(ytab_h.at[gi2d.at[a]], rows0, sem0)
                hb = pltpu.async_copy(ytab_h.at[gi2d.at[b]], rows1, sem1)
                ha.wait()
                pltpu.sync_copy(rows0, acc.at[dj2d.at[a]], add=True)
                hb.wait()
                pltpu.sync_copy(rows1, acc.at[dj2d.at[b]], add=True)
                return carry
            lax.fori_loop(0, nsc // 2, pbody, 0)

        plsc.subcore_barrier()
        pltpu.sync_copy(acc.at[pl.ds(sid * rpt, rpt)],
                        out_h.at[cid, pl.ds(sid * rpt, rpt)])

    return sck(ytab, src, dst, typ, zrows)


def kernel(node_embeddings, source_indices, dest_indices, edge_types,
           num_edges, W_e, b_e, W_ir, b_ir, W_hr, W_iz, b_iz, W_hz, W_in,
           b_in, W_hn, b_hn):
    n, h = node_embeddings.shape
    del num_edges

    rpt = ((n + _NS - 1) // _NS + 7) // 8 * 8
    npad = rpt * _NS

    w_cat = jnp.concatenate([W_e, W_ir, W_iz, W_in], axis=1)
    b_cat = jnp.concatenate([b_e, b_ir, b_iz, b_in]).reshape(-1, 1, h)

    y = _edge_transform(node_embeddings, w_cat, b_cat, nblk=1000)
    ytab = y.reshape(-1, h)

    zrows = jnp.zeros((rpt, h), jnp.float32)
    part = _sc_segment_sum(ytab, source_indices, dest_indices, edge_types,
                           zrows, n, npad, rpt)

    return _gru(part, y, W_hr, W_hz, W_hn, b_hn.reshape(1, h), nblk=1000)


# speedup vs baseline: 5.3512x; 1.0057x over previous
import functools

import jax
import jax.numpy as jnp
from jax import lax
from jax.experimental import pallas as pl
from jax.experimental.pallas import tpu as pltpu
from jax.experimental.pallas import tpu_sc as plsc

_H = 128
_NC = 2
_NS = 16
_CH = 128


def _dense_body(x_ref, w_ref, b_ref, o_ref):
    o_ref[0] = (
        jnp.dot(x_ref[...], w_ref[...], preferred_element_type=jnp.float32)
        + b_ref[0]
    )


def _edge_transform(x, w_cat, b_cat, nblk):
    n, h = x.shape
    g = w_cat.shape[1] // h
    ni = n // nblk
    return pl.pallas_call(
        _dense_body,
        grid=(ni, g),
        in_specs=[
            pl.BlockSpec((nblk, h), lambda i, t: (i, 0)),
            pl.BlockSpec((h, h), lambda i, t: (0, t)),
            pl.BlockSpec((1, 1, h), lambda i, t: (t, 0, 0)),
        ],
        out_specs=pl.BlockSpec((1, nblk, h), lambda i, t: (t, i, 0)),
        out_shape=jax.ShapeDtypeStruct((g, n, h), jnp.float32),
    )(x, w_cat, b_cat)


def _gru_body(part_ref, xr_ref, xz_ref, xn_ref, whr_ref, whz_ref, whn_ref,
              bhn_ref, o_ref):
    p = part_ref[0] + part_ref[1]
    r = jax.nn.sigmoid(
        xr_ref[0] + jnp.dot(p, whr_ref[...], preferred_element_type=jnp.float32))
    z = jax.nn.sigmoid(
        xz_ref[0] + jnp.dot(p, whz_ref[...], preferred_element_type=jnp.float32))
    nn = jnp.tanh(
        xn_ref[0]
        + r * (jnp.dot(p, whn_ref[...], preferred_element_type=jnp.float32)
               + bhn_ref[...]))
    o_ref[...] = (1.0 - z) * nn + z * p


def _gru(part, y, whr, whz, whn, bhn, nblk):
    npad = part.shape[1]
    n = y.shape[1]
    h = y.shape[2]
    ni = n // nblk
    return pl.pallas_call(
        _gru_body,
        grid=(ni,),
        in_specs=[
            pl.BlockSpec((2, nblk, h), lambda i: (0, i, 0)),
            pl.BlockSpec((1, nblk, h), lambda i: (6, i, 0)),
            pl.BlockSpec((1, nblk, h), lambda i: (7, i, 0)),
            pl.BlockSpec((1, nblk, h), lambda i: (8, i, 0)),
            pl.BlockSpec((h, h), lambda i: (0, 0)),
            pl.BlockSpec((h, h), lambda i: (0, 0)),
            pl.BlockSpec((h, h), lambda i: (0, 0)),
            pl.BlockSpec((1, h), lambda i: (0, 0)),
        ],
        out_specs=pl.BlockSpec((nblk, h), lambda i: (i, 0)),
        out_shape=jax.ShapeDtypeStruct((n, h), jnp.float32),
    )(part, y, y, y, whr, whz, whn, bhn)


def _sc_segment_sum(ytab, src, dst, typ, zrows, n_nodes, npad, rpt):
    e = src.shape[0]
    h = ytab.shape[1]
    nw = _NC * _NS
    epw = e // nw
    nsc = 20
    se = nsc * _CH
    nsup = -(-epw // se)

    mesh = plsc.VectorSubcoreMesh(core_axis_name="c", subcore_axis_name="s")

    @functools.partial(
        pl.kernel,
        mesh=mesh,
        out_type=jax.ShapeDtypeStruct((_NC, npad, h), jnp.float32),
        scratch_types=[
            pltpu.VMEM((se,), jnp.int32),
            pltpu.VMEM((se,), jnp.int32),
            pltpu.VMEM((se,), jnp.int32),
            pltpu.VMEM((nsc, _CH), jnp.int32),
            pltpu.VMEM((nsc, _CH), jnp.int32),
            pltpu.VMEM((_CH, h), jnp.float32),
            pltpu.VMEM((_CH, h), jnp.float32),
            pltpu.VMEM_SHARED((npad, h), jnp.float32),
            pltpu.SemaphoreType.DMA,
            pltpu.SemaphoreType.DMA,
            pltpu.SemaphoreType.DMA,
            pltpu.SemaphoreType.DMA,
        ],
    )
    def sck(ytab_h, src_h, dst_h, typ_h, z_h, out_h,
            sflat, tflat, dflat, gi2d, dj2d, rows0, rows1, acc,
            sem0, sem1, sem2, sem3):
        cid = lax.axis_index("c")
        sid = lax.axis_index("s")
        wid = cid * _NS + sid
        base = wid * epw

        pltpu.sync_copy(z_h, acc.at[pl.ds(sid * rpt, rpt)])
        plsc.subcore_barrier()

        zero16 = jnp.zeros((16,), jnp.int32)
        junk16 = jnp.full((16,), n_nodes, jnp.int32)

        for u in range(nsup):
            valid = min(se, epw - u * se)
            pltpu.sync_copy(src_h.at[pl.ds(base + u * se, valid)],
                            sflat.at[pl.ds(0, valid)])
            pltpu.sync_copy(typ_h.at[pl.ds(base + u * se, valid)],
                            tflat.at[pl.ds(0, valid)])
            pltpu.sync_copy(dst_h.at[pl.ds(base + u * se, valid)],
                            dflat.at[pl.ds(0, valid)])

            def cbody(j, carry):
                for k in range(_CH // 16):
                    off = j * _CH + k * 16
                    s = sflat[pl.ds(off, 16)]
                    t = tflat[pl.ds(off, 16)]
                    gi2d[j, pl.ds(k * 16, 16)] = t * n_nodes + s
                    dj2d[j, pl.ds(k * 16, 16)] = dflat[pl.ds(off, 16)]
                return carry
            lax.fori_loop(0, nsc, cbody, 0)

            for m in range(valid // 16, se // 16):
                j, k = m // (_CH // 16), m % (_CH // 16)
                gi2d[j, pl.ds(k * 16, 16)] = zero16
                dj2d[j, pl.ds(k * 16, 16)] = junk16

            def pbody(i, carry):
                c0 = i * 4
                ga = pltpu.async_copy(ytab_h.at[gi2d.at[c0]], rows0, sem0)
                gb = pltpu.async_copy(ytab_h.at[gi2d.at[c0 + 1]], rows1, sem1)
                ga.wait()
                sa = pltpu.async_copy(rows0, acc.at[dj2d.at[c0]], sem2,
                                      add=True)
                gb.wait()
                sb = pltpu.async_copy(rows1, acc.at[dj2d.at[c0 + 1]], sem3,
                                      add=True)
                sa.wait()
                gc = pltpu.async_copy(ytab_h.at[gi2d.at[c0 + 2]], rows0, sem0)
                sb.wait()
                gd = pltpu.async_copy(ytab_h.at[gi2d.at[c0 + 3]], rows1, sem1)
                gc.wait()
                se_ = pltpu.async_copy(rows0, acc.at[dj2d.at[c0 + 2]], sem2,
                                       add=True)
                gd.wait()
                sf = pltpu.async_copy(rows1, acc.at[dj2d.at[c0 + 3]], sem3,
                                      add=True)
                se_.wait()
                sf.wait()
                return carry
            lax.fori_loop(0, nsc // 4, pbody, 0)

        plsc.subcore_barrier()
        pltpu.sync_copy(acc.at[pl.ds(sid * rpt, rpt)],
                        out_h.at[cid, pl.ds(sid * rpt, rpt)])

    return sck(ytab, src, dst, typ, zrows)


def kernel(node_embeddings, source_indices, dest_indices, edge_types,
           num_edges, W_e, b_e, W_ir, b_ir, W_hr, W_iz, b_iz, W_hz, W_in,
           b_in, W_hn, b_hn):
    n, h = node_embeddings.shape
    del num_edges

    rpt = ((n + _NS - 1) // _NS + 7) // 8 * 8
    npad = rpt * _NS

    w_cat = jnp.concatenate([W_e, W_ir, W_iz, W_in], axis=1)
    b_cat = jnp.concatenate([b_e, b_ir, b_iz, b_in]).reshape(-1, 1, h)

    y = _edge_transform(node_embeddings, w_cat, b_cat, nblk=1000)
    ytab = y.reshape(-1, h)

    zrows = jnp.zeros((rpt, h), jnp.float32)
    part = _sc_segment_sum(ytab, source_indices, dest_indices, edge_types,
                           zrows, n, npad, rpt)

    return _gru(part, y, W_hr, W_hz, W_hn, b_hn.reshape(1, h), nblk=1000)
